# baseline (speedup 1.0000x reference)
import jax
import jax.numpy as jnp
from jax.experimental import pallas as pl

_CONF_THRES = 0.2
_IOU_THRES = 0.6
_MAX_DET = 300
_MAX_WH = 4096.0
_N = 5000
_ROWS = 8
_COLS = 640
_NPAD = _ROWS * _COLS
_NCLS = 80
_KSLOTS = 128
_T = 8


def _pp_kernel(pt_ref, out_ref):
    obj = pt_ref[4]

    def cls_body(c, carry):
        best, bcls = carry
        sc = obj * pt_ref[5 + c]
        better = sc > best
        return (jnp.where(better, sc, best), jnp.where(better, c, bcls))

    best0 = obj * pt_ref[5]
    bcls0 = jnp.zeros((_ROWS, _COLS), jnp.int32)
    best, bcls = jax.lax.fori_loop(1, _NCLS, cls_body, (best0, bcls0))
    scores = jnp.where(best > _CONF_THRES, best, 0.0)

    xc = pt_ref[0]
    yc = pt_ref[1]
    w = pt_ref[2]
    h = pt_ref[3]
    x1 = xc - w / 2.0
    y1 = yc - h / 2.0
    x2 = xc + w / 2.0
    y2 = yc + h / 2.0
    clsf = bcls.astype(jnp.float32)

    ridx = jax.lax.broadcasted_iota(jnp.int32, (_ROWS, _COLS), 0)
    cidx = jax.lax.broadcasted_iota(jnp.int32, (_ROWS, _COLS), 1)
    idx2 = ridx * _COLS + cidx
    lane = jax.lax.broadcasted_iota(jnp.int32, (1, 128), 1)
    krow = jax.lax.broadcasted_iota(jnp.int32, (_ROWS, _KSLOTS), 0)
    kcol = jax.lax.broadcasted_iota(jnp.int32, (_ROWS, _KSLOTS), 1)
    kslot = krow * _KSLOTS + kcol

    out_ref[...] = jnp.zeros_like(out_ref)

    zk = jnp.zeros((_ROWS, _KSLOTS), jnp.float32)
    m0 = jnp.max(scores)
    state0 = (scores, zk, zk, zk, zk, zk, jnp.int32(0), m0)
    selinit = jnp.full((_ROWS, _COLS), _T, jnp.int32)

    def cond(state):
        k = state[6]
        m = state[7]
        return (k < _MAX_DET) & (m > 0.0)

    def body(state):
        s, kx1, ky1, kx2, ky2, karea, k, _ = state

        def stream_body(t, carry):
            s, sel = carry
            mm = jnp.max(s, axis=1, keepdims=True)
            gm = jnp.max(mm, axis=0, keepdims=True)
            eq = s == gm
            im = jnp.where(eq, idx2, _NPAD)
            gi = jnp.min(jnp.min(im, axis=1, keepdims=True),
                         axis=0, keepdims=True)
            onehot = eq & (idx2 == gi)
            return (jnp.where(onehot, -1.0, s),
                    jnp.where(onehot, t, sel))

        s, sel = jax.lax.fori_loop(0, _T, stream_body, (s, selinit))
        m_next = jnp.max(s)

        for t in range(_T):
            oh = sel == t

            def pick(f):
                return jnp.sum(jnp.where(oh, f, 0.0))

            m = pick(scores)
            wx1 = pick(x1)
            wy1 = pick(y1)
            wx2 = pick(x2)
            wy2 = pick(y2)
            wcls = pick(clsf)
            woff = wcls * _MAX_WH
            cox1 = wx1 + woff
            coy1 = wy1 + woff
            cox2 = wx2 + woff
            coy2 = wy2 + woff
            ca2 = (cox2 - cox1) * (coy2 - coy1)

            ix1 = jnp.maximum(kx1, cox1)
            iy1 = jnp.maximum(ky1, coy1)
            ix2 = jnp.minimum(kx2, cox2)
            iy2 = jnp.minimum(ky2, coy2)
            inter = jnp.clip(ix2 - ix1, 0.0) * jnp.clip(iy2 - iy1, 0.0)
            iou = inter / (karea + ca2 - inter + 1e-9)
            keep = (jnp.max(iou) <= _IOU_THRES) & (m > 0.0) & (k < _MAX_DET)

            app = (kslot == k) & keep
            kx1 = jnp.where(app, cox1, kx1)
            ky1 = jnp.where(app, coy1, ky1)
            kx2 = jnp.where(app, cox2, kx2)
            ky2 = jnp.where(app, coy2, ky2)
            karea = jnp.where(app, ca2, karea)

            row = (
                jnp.where(lane == 0, wx1, 0.0)
                + jnp.where(lane == 1, wy1, 0.0)
                + jnp.where(lane == 2, wx2, 0.0)
                + jnp.where(lane == 3, wy2, 0.0)
                + jnp.where(lane == 4, m, 0.0)
                + jnp.where(lane == 5, wcls, 0.0)
            )
            out_ref[pl.ds(k, 1), :] = jnp.where(keep, row, 0.0)
            k = k + keep.astype(jnp.int32)

        return (s, kx1, ky1, kx2, ky2, karea, k, m_next)

    jax.lax.while_loop(cond, body, state0)


def kernel(preds, anchors, image_size):
    del anchors, image_size
    p = preds[0]
    p = jnp.pad(p, ((0, _NPAD - _N), (0, 0)))
    pt = p.T.reshape(85, _ROWS, _COLS)
    out = pl.pallas_call(
        _pp_kernel,
        out_shape=jax.ShapeDtypeStruct((_MAX_DET + 4, 128), jnp.float32),
    )(pt)
    return out[:_MAX_DET, :6].reshape(1, _MAX_DET, 6)


# speedup vs baseline: 1.2070x; 1.2070x over previous
import jax
import jax.numpy as jnp
from jax.experimental import pallas as pl

_CONF_THRES = 0.2
_IOU_THRES = 0.6
_MAX_DET = 300
_MAX_WH = 4096.0
_N = 5000
_ROWS = 8
_COLS = 640
_NPAD = _ROWS * _COLS
_NCLS = 80
_KSLOTS = 128


def _pp_kernel(pt_ref, out_ref):
    obj = pt_ref[4]

    def cls_body(c, carry):
        best, bcls = carry
        sc = obj * pt_ref[5 + c]
        better = sc > best
        return (jnp.where(better, sc, best), jnp.where(better, c, bcls))

    best0 = obj * pt_ref[5]
    bcls0 = jnp.zeros((_ROWS, _COLS), jnp.int32)
    best, bcls = jax.lax.fori_loop(1, _NCLS, cls_body, (best0, bcls0))
    scores = jnp.where(best > _CONF_THRES, best, 0.0)

    xc = pt_ref[0]
    yc = pt_ref[1]
    w = pt_ref[2]
    h = pt_ref[3]
    x1 = xc - w / 2.0
    y1 = yc - h / 2.0
    x2 = xc + w / 2.0
    y2 = yc + h / 2.0
    clsf = bcls.astype(jnp.float32)

    ridx = jax.lax.broadcasted_iota(jnp.int32, (_ROWS, _COLS), 0)
    cidx = jax.lax.broadcasted_iota(jnp.int32, (_ROWS, _COLS), 1)
    idx2 = ridx * _COLS + cidx
    lane = jax.lax.broadcasted_iota(jnp.int32, (1, 128), 1)
    krow = jax.lax.broadcasted_iota(jnp.int32, (_ROWS, _KSLOTS), 0)
    kcol = jax.lax.broadcasted_iota(jnp.int32, (_ROWS, _KSLOTS), 1)
    kslot = krow * _KSLOTS + kcol

    out_ref[...] = jnp.zeros_like(out_ref)

    zk = jnp.zeros((_ROWS, _KSLOTS), jnp.float32)
    m0 = jnp.max(scores)
    idx0 = jnp.min(jnp.where(scores == m0, idx2, _NPAD))
    state0 = (scores, zk, zk, zk, zk, zk, jnp.int32(0), m0, idx0)

    def cond(state):
        k = state[6]
        m = state[7]
        return (k < _MAX_DET) & (m > 0.0)

    def body(state):
        s, kx1, ky1, kx2, ky2, karea, k, m, idx = state

        onehot = idx2 == idx

        def pick(f):
            return jnp.sum(jnp.where(onehot, f, 0.0))

        wx1 = pick(x1)
        wy1 = pick(y1)
        wx2 = pick(x2)
        wy2 = pick(y2)
        wcls = pick(clsf)
        woff = wcls * _MAX_WH
        cox1 = wx1 + woff
        coy1 = wy1 + woff
        cox2 = wx2 + woff
        coy2 = wy2 + woff
        ca2 = (cox2 - cox1) * (coy2 - coy1)

        s = jnp.where(onehot, -1.0, s)
        m_next = jnp.max(s)
        idx_next = jnp.min(jnp.where(s == m_next, idx2, _NPAD))

        ix1 = jnp.maximum(kx1, cox1)
        iy1 = jnp.maximum(ky1, coy1)
        ix2 = jnp.minimum(kx2, cox2)
        iy2 = jnp.minimum(ky2, coy2)
        inter = jnp.clip(ix2 - ix1, 0.0) * jnp.clip(iy2 - iy1, 0.0)
        iou = inter / (karea + ca2 - inter + 1e-9)
        keep = jnp.max(iou) <= _IOU_THRES

        app = (kslot == k) & keep
        kx1 = jnp.where(app, cox1, kx1)
        ky1 = jnp.where(app, coy1, ky1)
        kx2 = jnp.where(app, cox2, kx2)
        ky2 = jnp.where(app, coy2, ky2)
        karea = jnp.where(app, ca2, karea)

        row = (
            jnp.where(lane == 0, wx1, 0.0)
            + jnp.where(lane == 1, wy1, 0.0)
            + jnp.where(lane == 2, wx2, 0.0)
            + jnp.where(lane == 3, wy2, 0.0)
            + jnp.where(lane == 4, m, 0.0)
            + jnp.where(lane == 5, wcls, 0.0)
        )
        out_ref[pl.ds(k, 1), :] = jnp.where(keep, row, 0.0)
        k = k + keep.astype(jnp.int32)

        return (s, kx1, ky1, kx2, ky2, karea, k, m_next, idx_next)

    jax.lax.while_loop(cond, body, state0)


def kernel(preds, anchors, image_size):
    del anchors, image_size
    p = preds[0]
    p = jnp.pad(p, ((0, _NPAD - _N), (0, 0)))
    pt = p.T.reshape(85, _ROWS, _COLS)
    out = pl.pallas_call(
        _pp_kernel,
        out_shape=jax.ShapeDtypeStruct((_MAX_DET + 4, 128), jnp.float32),
    )(pt)
    return out[:_MAX_DET, :6].reshape(1, _MAX_DET, 6)
